# baseline (speedup 1.0000x reference)
import functools

import jax
import jax.numpy as jnp
from jax import lax
from jax.experimental import pallas as pl
from jax.experimental.pallas import tpu as pltpu
from jax.experimental.pallas import tpu_sc as plsc

_CLASS_NUM = 100000
_CHANNEL = 128
_SIZE = 4
_BATCH = 1024
_C = 20
_PAIRS = _BATCH * _C
_XROWS = _C * _SIZE


def _make_sc_select():
    info = plsc.get_sparse_core_info()
    nw = info.num_cores * info.num_subcores
    rows_per_w = _PAIRS // nw
    b_per_w = _BATCH // nw
    bc = 4
    n_chunks = b_per_w // bc
    crows = bc * _C
    mesh = plsc.VectorSubcoreMesh(core_axis_name="c", subcore_axis_name="s")

    @functools.partial(
        pl.kernel,
        mesh=mesh,
        out_type=jax.ShapeDtypeStruct((_BATCH, _SIZE, _XROWS, _CHANNEL),
                                      jnp.float32),
        scratch_types=[
            pltpu.VMEM((rows_per_w,), jnp.int32),
            pltpu.VMEM((2, crows, _CHANNEL), jnp.float32),
            pltpu.VMEM((2, bc, _XROWS, _CHANNEL), jnp.float32),
            pltpu.SemaphoreType.DMA,
            pltpu.SemaphoreType.DMA,
        ],
    )
    def sc_select(table_hbm, lab_hbm, out_hbm, idx_v, rows_v, exp_v, gsem, osem):
        wid = lax.axis_index("s") * info.num_cores + lax.axis_index("c")
        ib = wid * b_per_w
        pltpu.sync_copy(lab_hbm.at[pl.ds(wid * rows_per_w, rows_per_w)], idx_v)

        def gather(cc, buf):
            return pltpu.make_async_copy(
                table_hbm.at[idx_v.at[pl.ds(cc * crows, crows)]],
                rows_v.at[buf],
                gsem,
            )

        def row_copies(cc, buf, b2):
            return [
                pltpu.make_async_copy(
                    exp_v.at[buf, b2],
                    out_hbm.at[ib + cc * bc + b2, s],
                    osem,
                )
                for s in range(_SIZE)
            ]

        def out_copies(cc, buf):
            return [cp for b2 in range(bc) for cp in row_copies(cc, buf, b2)]

        gather(0, 0).start()

        def group_body(it, _):
            cc0 = it * 2
            for b in range(2):
                cc = cc0 + b

                @pl.when(cc + 1 < n_chunks)
                def _prefetch():
                    gather(cc + 1, 1 - b).start()

                gather(cc, b).wait()

                @pl.when(cc >= 2)
                def _drain():
                    for cp in out_copies(cc - 2, b):
                        cp.wait()

                for b2 in range(bc):
                    def expand_row(j, _, b2=b2):
                        for l in range(_CHANNEL // 16):
                            v = rows_v[b, b2 * _C + j, pl.ds(l * 16, 16)]
                            for t in range(_SIZE):
                                exp_v[b, b2, j * _SIZE + t,
                                      pl.ds(l * 16, 16)] = v
                        return 0

                    lax.fori_loop(0, _C, expand_row, 0, unroll=2)
                    for cp in row_copies(cc, b, b2):
                        cp.start()
            return 0

        lax.fori_loop(0, n_chunks // 2, group_body, 0)

        for cc in (n_chunks - 2, n_chunks - 1):
            for cp in out_copies(cc, cc % 2):
                cp.wait()

    return sc_select


_SC_SELECT = _make_sc_select()


def kernel(labels, TextEmbeddings):
    table = TextEmbeddings.reshape(_CLASS_NUM, _CHANNEL)
    lab_flat = labels.reshape(_PAIRS)
    out4 = _SC_SELECT(table, lab_flat)
    return jnp.transpose(out4, (0, 3, 1, 2))
